# baseline (speedup 1.0000x reference)
import jax
import jax.numpy as jnp
from jax.experimental import pallas as pl

N = 10000; E = 320000; IN_F = 128; HID = 128; NC = 2; RATIO = 0.5; B = 8


def _gcn(x, src, dst, emask, nmask, W, b):
    deg = jnp.zeros(N, jnp.float32).at[dst].add(emask) + nmask
    dis = jnp.where(deg > 0, 1.0 / jnp.sqrt(jnp.maximum(deg, 1e-12)), 0.0)
    h = x @ W
    norm = dis[src] * dis[dst] * emask
    agg = jnp.zeros((N, h.shape[1]), jnp.float32).at[dst].add(h[src] * norm[:, None])
    agg = agg + h * (dis * dis * nmask)[:, None]
    return (agg + b) * nmask[:, None]


def _graphconv_score(x, src, dst, emask, Wrel, brel, Wroot):
    agg = jnp.zeros((N, x.shape[1]), jnp.float32).at[dst].add(x[src] * emask[:, None])
    return (agg @ Wrel + brel + x @ Wroot)[:, 0]


def _sag_pool(x, src, dst, emask, nmask, batch, Wrel, brel, Wroot):
    score = _graphconv_score(x, src, dst, emask, Wrel, brel, Wroot)
    score_m = jnp.where(nmask > 0, score, -jnp.inf)
    alive = jnp.zeros(B, jnp.float32).at[batch].add(nmask)
    k = jnp.ceil(RATIO * alive)
    order = jnp.lexsort((-score_m, batch))
    gtotal = jnp.bincount(batch, length=B)
    start = jnp.concatenate([jnp.zeros(1, gtotal.dtype), jnp.cumsum(gtotal)[:-1]])
    bs = batch[order]
    rank = jnp.arange(N) - start[bs]
    keep_sorted = (rank < k[bs]).astype(jnp.float32)
    keep = jnp.zeros(N, jnp.float32).at[order].set(keep_sorted)
    xg = x * jnp.tanh(score)[:, None] * keep[:, None]
    emask2 = emask * keep[src] * keep[dst]
    return xg, keep, emask2


def _pools(x, batch, nmask):
    cnt = jnp.zeros(B, jnp.float32).at[batch].add(nmask)
    s = jnp.zeros((B, x.shape[1]), jnp.float32).at[batch].add(x * nmask[:, None])
    mean = s / jnp.maximum(cnt, 1.0)[:, None]
    xm = jnp.where(nmask[:, None] > 0, x, -jnp.inf)
    mx = jnp.full((B, x.shape[1]), -jnp.inf, jnp.float32).at[batch].max(xm)
    mx = jnp.where(jnp.isfinite(mx), mx, 0.0)
    return jnp.concatenate([mean, mx], axis=1)


def _copy_kernel(x_ref, o_ref):
    o_ref[...] = x_ref[...]


def kernel(x, edge_index, batch, params):
    P = params
    src, dst = edge_index[0], edge_index[1]
    nmask = jnp.ones(N, jnp.float32)
    emask = jnp.ones(E, jnp.float32)
    reads = []
    h = x
    for j in range(1, 5):
        h = jax.nn.relu(_gcn(h, src, dst, emask, nmask, P['W%d' % j], P['b%d' % j]))
        h, nmask, emask = _sag_pool(h, src, dst, emask, nmask, batch,
                                    P['p%dWrel' % j], P['p%dbrel' % j], P['p%dWroot' % j])
        reads.append(_pools(h, batch, nmask))
    r = reads[0] + reads[1] + reads[2] + reads[3]
    r = pl.pallas_call(_copy_kernel, out_shape=jax.ShapeDtypeStruct(r.shape, r.dtype))(r)
    z = jax.nn.relu(r @ P['L1W'] + P['L1b'])
    z = jax.nn.relu(z @ P['L2W'] + P['L2b'])
    logits = z @ P['L3W'] + P['L3b']
    return logits, jax.nn.softmax(logits, axis=1)


# speedup vs baseline: 1.2937x; 1.2937x over previous
import functools

import jax
import jax.numpy as jnp
from jax import lax
from jax.experimental import pallas as pl
from jax.experimental.pallas import tpu as pltpu
from jax.experimental.pallas import tpu_sc as plsc

N = 10000
E = 320000
IN_F = 128
HID = 128
NCLS = 2
B = 8

NPAD = 10240
NR = NPAD // 128
EPAD = 327680
ER = EPAD // 128
NSC = 2
NSUB = 16
NW = NSC * NSUB
EW = EPAD // NW
EW_R = EW // 128
NTILE = NPAD // NSUB
DUMMY = NPAD - 1

_MESH = plsc.VectorSubcoreMesh(core_axis_name="c", subcore_axis_name="s")


def _zero_vec(ref, n):
  z = jnp.zeros((16,), ref.dtype)

  def body(i, _):
    ref[pl.ds(i * 16, 16)] = z
    return 0

  lax.fori_loop(0, n // 16, body, 0)


def _sc_edge_update(src_ref, dst_ref, keep_ref, srcout_ref, degpart_ref,
                    src_v, didx_v, ks_v, kd_v, upd_v, out_v, zvec_v,
                    sem, sh_keep, sh_deg):
  cid = lax.axis_index("c")
  sid = lax.axis_index("s")
  wid = sid * NSC + cid
  ebase = wid * EW
  rbase = wid * EW_R
  nt0 = sid * NTILE

  pltpu.sync_copy(src_ref.at[pl.ds(ebase, EW)], src_v)
  pltpu.sync_copy(dst_ref.at[pl.ds(rbase, EW_R)], didx_v)

  pltpu.sync_copy(keep_ref.at[pl.ds(nt0, NTILE)], sh_keep.at[pl.ds(nt0, NTILE)])
  _zero_vec(zvec_v, NTILE)
  pltpu.sync_copy(zvec_v, sh_deg.at[pl.ds(nt0, NTILE)])
  plsc.subcore_barrier()

  dummy16 = jnp.full((16,), DUMMY, jnp.int32)

  def body(r, _):
    pltpu.async_copy(sh_keep.at[src_v.at[pl.ds(r * 128, 128)]], ks_v,
                     sem).wait()
    pltpu.async_copy(sh_keep.at[didx_v.at[r]], kd_v, sem).wait()
    for c in range(8):
      s16 = src_v[pl.ds(r * 128 + c * 16, 16)]
      w = ks_v[pl.ds(c * 16, 16)] * kd_v[pl.ds(c * 16, 16)]
      out_v[pl.ds(r * 128 + c * 16, 16)] = jnp.where(w > 0.0, s16, dummy16)
      upd_v[r, pl.ds(c * 16, 16)] = w
    pltpu.sync_copy(upd_v.at[r], sh_deg.at[didx_v.at[r]], add=True)
    return 0

  lax.fori_loop(0, EW_R, body, 0)

  pltpu.sync_copy(out_v, srcout_ref.at[pl.ds(ebase, EW)])
  plsc.subcore_barrier()
  pltpu.sync_copy(sh_deg.at[pl.ds(nt0, NTILE)],
                  degpart_ref.at[cid, pl.ds(nt0, NTILE)])


_edge_update = pl.kernel(
    _sc_edge_update,
    out_type=[
        jax.ShapeDtypeStruct((EPAD,), jnp.int32),
        jax.ShapeDtypeStruct((NSC, NPAD), jnp.float32),
    ],
    mesh=_MESH,
    scratch_types=[
        pltpu.VMEM((EW,), jnp.int32),
        pltpu.VMEM((EW_R, 128), jnp.int32),
        pltpu.VMEM((128,), jnp.float32),
        pltpu.VMEM((128,), jnp.float32),
        pltpu.VMEM((EW_R, 128), jnp.float32),
        pltpu.VMEM((EW,), jnp.int32),
        pltpu.VMEM((NTILE,), jnp.float32),
        pltpu.SemaphoreType.DMA,
        pltpu.VMEM_SHARED((NPAD,), jnp.float32),
        pltpu.VMEM_SHARED((NPAD,), jnp.float32),
    ],
)


def _sc_edge_score(src_ref, dst_ref, t_ref, spart_ref,
                   src_v, didx_v, tg_v, zvec_v, sem, sh_t, sh_s):
  cid = lax.axis_index("c")
  sid = lax.axis_index("s")
  wid = sid * NSC + cid
  ebase = wid * EW
  rbase = wid * EW_R
  nt0 = sid * NTILE

  pltpu.sync_copy(src_ref.at[pl.ds(ebase, EW)], src_v)
  pltpu.sync_copy(dst_ref.at[pl.ds(rbase, EW_R)], didx_v)

  pltpu.sync_copy(t_ref.at[pl.ds(nt0, NTILE)], sh_t.at[pl.ds(nt0, NTILE)])
  _zero_vec(zvec_v, NTILE)
  pltpu.sync_copy(zvec_v, sh_s.at[pl.ds(nt0, NTILE)])
  plsc.subcore_barrier()

  def body(r, _):
    pltpu.async_copy(sh_t.at[src_v.at[pl.ds(r * 128, 128)]], tg_v,
                     sem).wait()
    pltpu.sync_copy(tg_v, sh_s.at[didx_v.at[r]], add=True)
    return 0

  lax.fori_loop(0, EW_R, body, 0)

  plsc.subcore_barrier()
  pltpu.sync_copy(sh_s.at[pl.ds(nt0, NTILE)],
                  spart_ref.at[cid, pl.ds(nt0, NTILE)])


_edge_score = pl.kernel(
    _sc_edge_score,
    out_type=[jax.ShapeDtypeStruct((NSC, NPAD), jnp.float32)],
    mesh=_MESH,
    scratch_types=[
        pltpu.VMEM((EW,), jnp.int32),
        pltpu.VMEM((EW_R, 128), jnp.int32),
        pltpu.VMEM((128,), jnp.float32),
        pltpu.VMEM((NTILE,), jnp.float32),
        pltpu.SemaphoreType.DMA,
        pltpu.VMEM_SHARED((NPAD,), jnp.float32),
        pltpu.VMEM_SHARED((NPAD,), jnp.float32),
    ],
)


def _sc_gather_agg(hs_ref, src_ref, dst_ref, mpart_ref,
                   sidx_v, didx_v, rows_v, sem, sh_acc):
  cid = lax.axis_index("c")
  sid = lax.axis_index("s")
  wid = sid * NSC + cid
  ebase = wid * EW
  rbase = wid * EW_R

  pltpu.sync_copy(src_ref.at[pl.ds(ebase, EW)], sidx_v)
  pltpu.sync_copy(dst_ref.at[pl.ds(rbase, EW_R)], didx_v)

  def zb(r, _):
    for c in range(8):
      rows_v[r, pl.ds(c * 16, 16)] = jnp.zeros((16,), jnp.float32)
    return 0

  lax.fori_loop(0, 128, zb, 0)
  for z in range(NTILE // 128):
    pltpu.sync_copy(rows_v, sh_acc.at[pl.ds(sid * NTILE + z * 128, 128)])
  plsc.subcore_barrier()

  def body(j, _):
    pltpu.async_copy(hs_ref.at[sidx_v.at[pl.ds(j * 128, 128)]],
                     rows_v, sem).wait()
    pltpu.sync_copy(rows_v, sh_acc.at[didx_v.at[j]], add=True)
    return 0

  lax.fori_loop(0, EW_R, body, 0)
  plsc.subcore_barrier()

  for z in range(NTILE // 128):
    r0 = sid * NTILE + z * 128
    pltpu.sync_copy(sh_acc.at[pl.ds(r0, 128)],
                    mpart_ref.at[cid, pl.ds(r0, 128)])


_gather_agg = pl.kernel(
    _sc_gather_agg,
    out_type=[jax.ShapeDtypeStruct((NSC, NPAD, 128), jnp.float32)],
    mesh=_MESH,
    scratch_types=[
        pltpu.VMEM((EW,), jnp.int32),
        pltpu.VMEM((EW_R, 128), jnp.int32),
        pltpu.VMEM((128, 128), jnp.float32),
        pltpu.SemaphoreType.DMA,
        pltpu.VMEM_SHARED((NPAD, 128), jnp.float32),
    ],
)


_BLK = 1024


def _tc1_body(x_ref, w_ref, degpart_ref, nmask_ref, hs_ref, dis_ref):
  deg = degpart_ref[0] + degpart_ref[1] + nmask_ref[...]
  dis = jnp.where(deg > 0.0, lax.rsqrt(jnp.maximum(deg, 1e-12)), 0.0)
  h = jnp.dot(x_ref[...], w_ref[...], preferred_element_type=jnp.float32)
  hs_ref[...] = h * dis
  dis_ref[...] = dis


def _tc1(x, w, degpart, nmask):
  grid = NPAD // _BLK
  return pl.pallas_call(
      _tc1_body,
      grid=(grid,),
      in_specs=[
          pl.BlockSpec((_BLK, 128), lambda i: (i, 0)),
          pl.BlockSpec((128, 128), lambda i: (0, 0)),
          pl.BlockSpec((NSC, _BLK, 1), lambda i: (0, i, 0)),
          pl.BlockSpec((_BLK, 1), lambda i: (i, 0)),
      ],
      out_specs=[
          pl.BlockSpec((_BLK, 128), lambda i: (i, 0)),
          pl.BlockSpec((_BLK, 1), lambda i: (i, 0)),
      ],
      out_shape=[
          jax.ShapeDtypeStruct((NPAD, 128), jnp.float32),
          jax.ShapeDtypeStruct((NPAD, 1), jnp.float32),
      ],
  )(x, w, degpart, nmask)


def _tc2_body(m_ref, hs_ref, dis_ref, nmask_ref, b_ref, wrel_ref, wroot_ref,
              hout_ref, t_ref, u_ref):
  nm = nmask_ref[...]
  agg = (m_ref[0] + m_ref[1] + hs_ref[...] * nm) * dis_ref[...] + b_ref[...]
  h = nm * jnp.maximum(agg, 0.0)
  hout_ref[...] = h
  t_ref[...] = jnp.sum(h * wrel_ref[...], axis=1, keepdims=True)
  u_ref[...] = jnp.sum(h * wroot_ref[...], axis=1, keepdims=True)


def _tc2(mpart, hs, dis, nmask, b, wrel, wroot):
  grid = NPAD // _BLK
  return pl.pallas_call(
      _tc2_body,
      grid=(grid,),
      in_specs=[
          pl.BlockSpec((NSC, _BLK, 128), lambda i: (0, i, 0)),
          pl.BlockSpec((_BLK, 128), lambda i: (i, 0)),
          pl.BlockSpec((_BLK, 1), lambda i: (i, 0)),
          pl.BlockSpec((_BLK, 1), lambda i: (i, 0)),
          pl.BlockSpec((1, 128), lambda i: (0, 0)),
          pl.BlockSpec((1, 128), lambda i: (0, 0)),
          pl.BlockSpec((1, 128), lambda i: (0, 0)),
      ],
      out_specs=[
          pl.BlockSpec((_BLK, 128), lambda i: (i, 0)),
          pl.BlockSpec((_BLK, 1), lambda i: (i, 0)),
          pl.BlockSpec((_BLK, 1), lambda i: (i, 0)),
      ],
      out_shape=[
          jax.ShapeDtypeStruct((NPAD, 128), jnp.float32),
          jax.ShapeDtypeStruct((NPAD, 1), jnp.float32),
          jax.ShapeDtypeStruct((NPAD, 1), jnp.float32),
      ],
  )(mpart, hs, dis, nmask, b, wrel, wroot)


def _tc3_body(spart_ref, u_ref, brel_ref, batch_ref, nmask_ref,
              keep_ref, g_ref):
  score = spart_ref[0] + spart_ref[1] + brel_ref[...] + u_ref[...]
  nm = nmask_ref[...]
  batch = batch_ref[...]

  neg_inf = jnp.float32(-jnp.inf)
  sm = jnp.where(nm > 0.0, score, neg_inf)
  ib = lax.bitcast_convert_type(sm, jnp.int32)
  sign_bit = jnp.int32(-2147483648)
  ku = jnp.where(ib >= 0, ib ^ sign_bit, ~ib).astype(jnp.uint32)

  rows = lax.broadcasted_iota(jnp.int32, (NR, 128), 0)
  cols = lax.broadcasted_iota(jnp.int32, (NR, 128), 1)
  mflip = (NPAD - 1 - (rows * 128 + cols)).astype(jnp.uint32)

  keep_acc = jnp.zeros((NR, 128), jnp.float32)
  one_u = jnp.uint32(1)

  for b in range(B):
    mb = batch == b
    alive = jnp.sum(jnp.where(mb, nm, 0.0))
    k = jnp.floor((alive + 1.0) * 0.5)

    def bit_step(i, v):
      cand = v | (one_u << (jnp.uint32(31) - i.astype(jnp.uint32)))
      cnt = jnp.sum(jnp.where(mb & (ku >= cand), 1.0, 0.0))
      return jnp.where(cnt >= k, cand, v)

    v = lax.fori_loop(0, 32, bit_step, jnp.uint32(0))

    cnt_gt = jnp.sum(jnp.where(mb & (ku > v), 1.0, 0.0))
    tie_need = k - cnt_gt
    eq = mb & (ku == v)

    def tie_step(i, t):
      cand = t | (one_u << (jnp.uint32(13) - i.astype(jnp.uint32)))
      cnt = jnp.sum(jnp.where(eq & (mflip >= cand), 1.0, 0.0))
      return jnp.where(cnt >= tie_need, cand, t)

    tau = lax.fori_loop(0, 14, tie_step, jnp.uint32(0))

    keep_b = jnp.where(mb & (ku > v), 1.0, 0.0)
    keep_b = keep_b + jnp.where(eq & (mflip >= tau) & (tie_need > 0.0),
                                1.0, 0.0)
    keep_acc = keep_acc + jnp.where(k > 0.0, keep_b, 0.0)

  keep_ref[...] = keep_acc
  g_ref[...] = jnp.tanh(score) * keep_acc


def _tc3(spart, u2, brel, batch2, nmask2):
  return pl.pallas_call(
      _tc3_body,
      out_shape=[
          jax.ShapeDtypeStruct((NR, 128), jnp.float32),
          jax.ShapeDtypeStruct((NR, 128), jnp.float32),
      ],
  )(spart, u2, brel, batch2, nmask2)


def _tc4_body(hout_ref, g_ref, keep_ref, batch_ref, xg_ref, mean_ref, mx_ref):
  xg = hout_ref[...] * g_ref[...]
  xg_ref[...] = xg
  keep = keep_ref[...]
  batch = batch_ref[...]

  onehot = (batch == lax.broadcasted_iota(jnp.int32, (1, B), 1)).astype(
      jnp.float32)
  dn = (((0,), (0,)), ((), ()))
  sums = lax.dot_general(onehot, xg, dn, preferred_element_type=jnp.float32)
  cnt = lax.dot_general(onehot, keep, dn, preferred_element_type=jnp.float32)
  mean_ref[...] = sums / jnp.maximum(cnt, 1.0)

  neg_inf = jnp.float32(-jnp.inf)
  xm = jnp.where(keep > 0.0, xg, neg_inf)
  mx_rows = []
  for b in range(B):
    col = jnp.max(jnp.where(batch == b, xm, neg_inf), axis=0, keepdims=True)
    mx_rows.append(jnp.where(jnp.isfinite(col), col, 0.0))
  mx_ref[...] = jnp.concatenate(mx_rows, axis=0)


def _tc4(hout, g, keep, batch):
  return pl.pallas_call(
      _tc4_body,
      out_shape=[
          jax.ShapeDtypeStruct((NPAD, 128), jnp.float32),
          jax.ShapeDtypeStruct((B, 128), jnp.float32),
          jax.ShapeDtypeStruct((B, 128), jnp.float32),
      ],
  )(hout, g, keep, batch)


def _tc5_body(mean_ref, mx_ref, w1_ref, b1_ref, w2_ref, b2_ref, w3_ref,
              b3_ref, logits_ref, probs_ref):
  mean = mean_ref[0] + mean_ref[1] + mean_ref[2] + mean_ref[3]
  mx = mx_ref[0] + mx_ref[1] + mx_ref[2] + mx_ref[3]
  r = jnp.concatenate([mean, mx], axis=1)
  z = jnp.maximum(
      jnp.dot(r, w1_ref[...], preferred_element_type=jnp.float32)
      + b1_ref[...], 0.0)
  z = jnp.maximum(
      jnp.dot(z, w2_ref[...], preferred_element_type=jnp.float32)
      + b2_ref[...], 0.0)
  lg = jnp.dot(z, w3_ref[...], preferred_element_type=jnp.float32) \
      + b3_ref[...]
  logits_ref[...] = lg
  m = jnp.max(lg, axis=1, keepdims=True)
  e = jnp.exp(lg - m)
  probs_ref[...] = e / jnp.sum(e, axis=1, keepdims=True)


def _tc5(means, maxes, w1, b1, w2, b2, w3, b3):
  return pl.pallas_call(
      _tc5_body,
      out_shape=[
          jax.ShapeDtypeStruct((B, NCLS), jnp.float32),
          jax.ShapeDtypeStruct((B, NCLS), jnp.float32),
      ],
  )(means, maxes, w1, b1, w2, b2, w3, b3)


def kernel(x, edge_index, batch, params):
  P = params
  src = edge_index[0].astype(jnp.int32)
  dst = edge_index[1].astype(jnp.int32)

  xp = jnp.pad(x.astype(jnp.float32), ((0, NPAD - N), (0, 0)))
  src_m = jnp.concatenate(
      [src, jnp.full((EPAD - E,), DUMMY, jnp.int32)])
  dst2d = jnp.concatenate(
      [dst, jnp.zeros((EPAD - E,), jnp.int32)]).reshape(ER, 128)
  batchp = jnp.pad(batch.astype(jnp.int32), (0, NPAD - N),
                   constant_values=B - 1)
  batch2 = batchp.reshape(NR, 128)
  batchc = batchp.reshape(NPAD, 1)
  keep = jnp.pad(jnp.ones((N,), jnp.float32), (0, NPAD - N))

  means = []
  maxes = []
  h = xp
  for j in (1, 2, 3, 4):
    src_m, degpart = _edge_update(src_m, dst2d, keep)
    nmaskc = keep.reshape(NPAD, 1)
    hs, dis = _tc1(h, P['W%d' % j], degpart.reshape(NSC, NPAD, 1), nmaskc)
    (mpart,) = _gather_agg(hs, src_m, dst2d)
    hout, t, u = _tc2(
        mpart, hs, dis, nmaskc,
        P['b%d' % j].reshape(1, 128),
        P['p%dWrel' % j].reshape(1, 128),
        P['p%dWroot' % j].reshape(1, 128))
    (spart,) = _edge_score(src_m, dst2d, t.reshape(NPAD))
    keep2, g2 = _tc3(
        spart.reshape(NSC, NR, 128),
        u.reshape(NR, 128),
        P['p%dbrel' % j].reshape(1, 1),
        batch2,
        keep.reshape(NR, 128))
    keep = keep2.reshape(NPAD)
    xg, mean_j, mx_j = _tc4(hout, g2.reshape(NPAD, 1), keep.reshape(NPAD, 1),
                            batchc)
    means.append(mean_j)
    maxes.append(mx_j)
    h = xg

  logits, probs = _tc5(
      jnp.stack(means), jnp.stack(maxes),
      P['L1W'], P['L1b'].reshape(1, HID),
      P['L2W'], P['L2b'].reshape(1, HID // 2),
      P['L3W'], P['L3b'].reshape(1, NCLS))
  return logits, probs


# speedup vs baseline: 31.0385x; 23.9921x over previous
import functools

import jax
import jax.numpy as jnp
from jax import lax
from jax.experimental import pallas as pl
from jax.experimental.pallas import tpu as pltpu
from jax.experimental.pallas import tpu_sc as plsc

N = 10000
E = 320000
IN_F = 128
HID = 128
NCLS = 2
B = 8

NPAD = 10240
NR = NPAD // 128
EPAD = 327680
ER = EPAD // 128
NSC = 2
NSUB = 16
NW = NSC * NSUB
EW = EPAD // NW
EW_R = EW // 128
NTILE = NPAD // NSUB
DUMMY = NPAD - 1

_MESH = plsc.VectorSubcoreMesh(core_axis_name="c", subcore_axis_name="s")


def _zero_vec(ref, n):
  z = jnp.zeros((16,), ref.dtype)

  def body(i, _):
    ref[pl.ds(i * 16, 16)] = z
    return 0

  lax.fori_loop(0, n // 16, body, 0)


def _sc_edge_update(src_ref, dst_ref, keep_ref, srcout_ref, degpart_ref,
                    src_v, didx_v, ks_v, kd_v, upd_v, out_v, zvec_v,
                    sem, sh_keep, sh_deg):
  cid = lax.axis_index("c")
  sid = lax.axis_index("s")
  wid = sid * NSC + cid
  ebase = wid * EW
  rbase = wid * EW_R
  nt0 = sid * NTILE

  pltpu.sync_copy(src_ref.at[pl.ds(ebase, EW)], src_v)
  pltpu.sync_copy(dst_ref.at[pl.ds(rbase, EW_R)], didx_v)

  pltpu.sync_copy(keep_ref.at[pl.ds(nt0, NTILE)], sh_keep.at[pl.ds(nt0, NTILE)])
  _zero_vec(zvec_v, NTILE)
  pltpu.sync_copy(zvec_v, sh_deg.at[pl.ds(nt0, NTILE)])
  plsc.subcore_barrier()

  def body(r, _):
    pltpu.async_copy(sh_keep.at[src_v.at[pl.ds(r * 128, 128)]], ks_v,
                     sem).wait()
    pltpu.async_copy(sh_keep.at[didx_v.at[r]], kd_v, sem).wait()
    for c in range(8):
      dummy16 = N + ((r * 128 + c * 16 + lax.iota(jnp.int32, 16))
                     % (NPAD - N))
      s16 = src_v[pl.ds(r * 128 + c * 16, 16)]
      w = ks_v[pl.ds(c * 16, 16)] * kd_v[pl.ds(c * 16, 16)]
      out_v[pl.ds(r * 128 + c * 16, 16)] = jnp.where(w > 0.0, s16, dummy16)
      upd_v[r, pl.ds(c * 16, 16)] = w
    pltpu.sync_copy(upd_v.at[r], sh_deg.at[didx_v.at[r]], add=True)
    return 0

  lax.fori_loop(0, EW_R, body, 0)

  pltpu.sync_copy(out_v, srcout_ref.at[pl.ds(ebase, EW)])
  plsc.subcore_barrier()
  pltpu.sync_copy(sh_deg.at[pl.ds(nt0, NTILE)],
                  degpart_ref.at[cid, pl.ds(nt0, NTILE)])


_edge_update = pl.kernel(
    _sc_edge_update,
    out_type=[
        jax.ShapeDtypeStruct((EPAD,), jnp.int32),
        jax.ShapeDtypeStruct((NSC, NPAD), jnp.float32),
    ],
    mesh=_MESH,
    scratch_types=[
        pltpu.VMEM((EW,), jnp.int32),
        pltpu.VMEM((EW_R, 128), jnp.int32),
        pltpu.VMEM((128,), jnp.float32),
        pltpu.VMEM((128,), jnp.float32),
        pltpu.VMEM((EW_R, 128), jnp.float32),
        pltpu.VMEM((EW,), jnp.int32),
        pltpu.VMEM((NTILE,), jnp.float32),
        pltpu.SemaphoreType.DMA,
        pltpu.VMEM_SHARED((NPAD,), jnp.float32),
        pltpu.VMEM_SHARED((NPAD,), jnp.float32),
    ],
)


def _sc_edge_score(src_ref, dst_ref, t_ref, spart_ref,
                   src_v, didx_v, tg_v, zvec_v, sem, sh_t, sh_s):
  cid = lax.axis_index("c")
  sid = lax.axis_index("s")
  wid = sid * NSC + cid
  ebase = wid * EW
  rbase = wid * EW_R
  nt0 = sid * NTILE

  pltpu.sync_copy(src_ref.at[pl.ds(ebase, EW)], src_v)
  pltpu.sync_copy(dst_ref.at[pl.ds(rbase, EW_R)], didx_v)

  pltpu.sync_copy(t_ref.at[pl.ds(nt0, NTILE)], sh_t.at[pl.ds(nt0, NTILE)])
  _zero_vec(zvec_v, NTILE)
  pltpu.sync_copy(zvec_v, sh_s.at[pl.ds(nt0, NTILE)])
  plsc.subcore_barrier()

  def body(r, _):
    pltpu.async_copy(sh_t.at[src_v.at[pl.ds(r * 128, 128)]], tg_v,
                     sem).wait()
    pltpu.sync_copy(tg_v, sh_s.at[didx_v.at[r]], add=True)
    return 0

  lax.fori_loop(0, EW_R, body, 0)

  plsc.subcore_barrier()
  pltpu.sync_copy(sh_s.at[pl.ds(nt0, NTILE)],
                  spart_ref.at[cid, pl.ds(nt0, NTILE)])


_edge_score = pl.kernel(
    _sc_edge_score,
    out_type=[jax.ShapeDtypeStruct((NSC, NPAD), jnp.float32)],
    mesh=_MESH,
    scratch_types=[
        pltpu.VMEM((EW,), jnp.int32),
        pltpu.VMEM((EW_R, 128), jnp.int32),
        pltpu.VMEM((128,), jnp.float32),
        pltpu.VMEM((NTILE,), jnp.float32),
        pltpu.SemaphoreType.DMA,
        pltpu.VMEM_SHARED((NPAD,), jnp.float32),
        pltpu.VMEM_SHARED((NPAD,), jnp.float32),
    ],
)


def _sc_gather_agg(hs_ref, src_ref, dst_ref, mpart_ref,
                   sidx_v, didx_v, rows_v, sem, sh_acc):
  cid = lax.axis_index("c")
  sid = lax.axis_index("s")
  wid = sid * NSC + cid
  ebase = wid * EW
  rbase = wid * EW_R

  pltpu.sync_copy(src_ref.at[pl.ds(ebase, EW)], sidx_v)
  pltpu.sync_copy(dst_ref.at[pl.ds(rbase, EW_R)], didx_v)

  def zb(r, _):
    for c in range(8):
      rows_v[r, pl.ds(c * 16, 16)] = jnp.zeros((16,), jnp.float32)
    return 0

  lax.fori_loop(0, 128, zb, 0)
  for z in range(NTILE // 128):
    pltpu.sync_copy(rows_v, sh_acc.at[pl.ds(sid * NTILE + z * 128, 128)])
  plsc.subcore_barrier()

  def body(j, _):
    pltpu.async_copy(hs_ref.at[sidx_v.at[pl.ds(j * 128, 128)]],
                     rows_v, sem).wait()
    pltpu.sync_copy(rows_v, sh_acc.at[didx_v.at[j]], add=True)
    return 0

  lax.fori_loop(0, EW_R, body, 0)
  plsc.subcore_barrier()

  for z in range(NTILE // 128):
    r0 = sid * NTILE + z * 128
    pltpu.sync_copy(sh_acc.at[pl.ds(r0, 128)],
                    mpart_ref.at[cid, pl.ds(r0, 128)])


_gather_agg = pl.kernel(
    _sc_gather_agg,
    out_type=[jax.ShapeDtypeStruct((NSC, NPAD, 128), jnp.float32)],
    mesh=_MESH,
    scratch_types=[
        pltpu.VMEM((EW,), jnp.int32),
        pltpu.VMEM((EW_R, 128), jnp.int32),
        pltpu.VMEM((128, 128), jnp.float32),
        pltpu.SemaphoreType.DMA,
        pltpu.VMEM_SHARED((NPAD, 128), jnp.float32),
    ],
)


_BLK = 1024


def _tc1_body(x_ref, w_ref, degpart_ref, nmask_ref, hs_ref, dis_ref):
  deg = degpart_ref[0] + degpart_ref[1] + nmask_ref[...]
  dis = jnp.where(deg > 0.0, lax.rsqrt(jnp.maximum(deg, 1e-12)), 0.0)
  h = jnp.dot(x_ref[...], w_ref[...], preferred_element_type=jnp.float32)
  hs_ref[...] = h * dis
  dis_ref[...] = dis


def _tc1(x, w, degpart, nmask):
  grid = NPAD // _BLK
  return pl.pallas_call(
      _tc1_body,
      grid=(grid,),
      in_specs=[
          pl.BlockSpec((_BLK, 128), lambda i: (i, 0)),
          pl.BlockSpec((128, 128), lambda i: (0, 0)),
          pl.BlockSpec((NSC, _BLK, 1), lambda i: (0, i, 0)),
          pl.BlockSpec((_BLK, 1), lambda i: (i, 0)),
      ],
      out_specs=[
          pl.BlockSpec((_BLK, 128), lambda i: (i, 0)),
          pl.BlockSpec((_BLK, 1), lambda i: (i, 0)),
      ],
      out_shape=[
          jax.ShapeDtypeStruct((NPAD, 128), jnp.float32),
          jax.ShapeDtypeStruct((NPAD, 1), jnp.float32),
      ],
  )(x, w, degpart, nmask)


def _tc2_body(m_ref, hs_ref, dis_ref, nmask_ref, b_ref, wrel_ref, wroot_ref,
              hout_ref, t_ref, u_ref):
  nm = nmask_ref[...]
  agg = (m_ref[0] + m_ref[1] + hs_ref[...] * nm) * dis_ref[...] + b_ref[...]
  h = nm * jnp.maximum(agg, 0.0)
  hout_ref[...] = h
  t_ref[...] = jnp.sum(h * wrel_ref[...], axis=1, keepdims=True)
  u_ref[...] = jnp.sum(h * wroot_ref[...], axis=1, keepdims=True)


def _tc2(mpart, hs, dis, nmask, b, wrel, wroot):
  grid = NPAD // _BLK
  return pl.pallas_call(
      _tc2_body,
      grid=(grid,),
      in_specs=[
          pl.BlockSpec((NSC, _BLK, 128), lambda i: (0, i, 0)),
          pl.BlockSpec((_BLK, 128), lambda i: (i, 0)),
          pl.BlockSpec((_BLK, 1), lambda i: (i, 0)),
          pl.BlockSpec((_BLK, 1), lambda i: (i, 0)),
          pl.BlockSpec((1, 128), lambda i: (0, 0)),
          pl.BlockSpec((1, 128), lambda i: (0, 0)),
          pl.BlockSpec((1, 128), lambda i: (0, 0)),
      ],
      out_specs=[
          pl.BlockSpec((_BLK, 128), lambda i: (i, 0)),
          pl.BlockSpec((_BLK, 1), lambda i: (i, 0)),
          pl.BlockSpec((_BLK, 1), lambda i: (i, 0)),
      ],
      out_shape=[
          jax.ShapeDtypeStruct((NPAD, 128), jnp.float32),
          jax.ShapeDtypeStruct((NPAD, 1), jnp.float32),
          jax.ShapeDtypeStruct((NPAD, 1), jnp.float32),
      ],
  )(mpart, hs, dis, nmask, b, wrel, wroot)


def _tc3_body(spart_ref, u_ref, brel_ref, batch_ref, nmask_ref,
              keep_ref, g_ref):
  score = spart_ref[0] + spart_ref[1] + brel_ref[...] + u_ref[...]
  nm = nmask_ref[...]
  batch = batch_ref[...]

  neg_inf = jnp.float32(-jnp.inf)
  sm = jnp.where(nm > 0.0, score, neg_inf)
  ib = lax.bitcast_convert_type(sm, jnp.int32)
  sign_bit = jnp.int32(-2147483648)
  ku = jnp.where(ib >= 0, ib ^ sign_bit, ~ib).astype(jnp.uint32)

  rows = lax.broadcasted_iota(jnp.int32, (NR, 128), 0)
  cols = lax.broadcasted_iota(jnp.int32, (NR, 128), 1)
  mflip = (NPAD - 1 - (rows * 128 + cols)).astype(jnp.uint32)

  keep_acc = jnp.zeros((NR, 128), jnp.float32)
  one_u = jnp.uint32(1)

  for b in range(B):
    mb = batch == b
    alive = jnp.sum(jnp.where(mb, nm, 0.0))
    k = jnp.floor((alive + 1.0) * 0.5)

    def bit_step(i, v):
      cand = v | (one_u << (jnp.uint32(31) - i.astype(jnp.uint32)))
      cnt = jnp.sum(jnp.where(mb & (ku >= cand), 1.0, 0.0))
      return jnp.where(cnt >= k, cand, v)

    v = lax.fori_loop(0, 32, bit_step, jnp.uint32(0))

    cnt_gt = jnp.sum(jnp.where(mb & (ku > v), 1.0, 0.0))
    tie_need = k - cnt_gt
    eq = mb & (ku == v)

    def tie_step(i, t):
      cand = t | (one_u << (jnp.uint32(13) - i.astype(jnp.uint32)))
      cnt = jnp.sum(jnp.where(eq & (mflip >= cand), 1.0, 0.0))
      return jnp.where(cnt >= tie_need, cand, t)

    tau = lax.fori_loop(0, 14, tie_step, jnp.uint32(0))

    keep_b = jnp.where(mb & (ku > v), 1.0, 0.0)
    keep_b = keep_b + jnp.where(eq & (mflip >= tau) & (tie_need > 0.0),
                                1.0, 0.0)
    keep_acc = keep_acc + jnp.where(k > 0.0, keep_b, 0.0)

  keep_ref[...] = keep_acc
  g_ref[...] = jnp.tanh(score) * keep_acc


def _tc3(spart, u2, brel, batch2, nmask2):
  return pl.pallas_call(
      _tc3_body,
      out_shape=[
          jax.ShapeDtypeStruct((NR, 128), jnp.float32),
          jax.ShapeDtypeStruct((NR, 128), jnp.float32),
      ],
  )(spart, u2, brel, batch2, nmask2)


def _tc4_body(hout_ref, g_ref, keep_ref, batch_ref, xg_ref, mean_ref, mx_ref):
  xg = hout_ref[...] * g_ref[...]
  xg_ref[...] = xg
  keep = keep_ref[...]
  batch = batch_ref[...]

  onehot = (batch == lax.broadcasted_iota(jnp.int32, (1, B), 1)).astype(
      jnp.float32)
  dn = (((0,), (0,)), ((), ()))
  sums = lax.dot_general(onehot, xg, dn, preferred_element_type=jnp.float32)
  cnt = lax.dot_general(onehot, keep, dn, preferred_element_type=jnp.float32)
  mean_ref[...] = sums / jnp.maximum(cnt, 1.0)

  neg_inf = jnp.float32(-jnp.inf)
  xm = jnp.where(keep > 0.0, xg, neg_inf)
  mx_rows = []
  for b in range(B):
    col = jnp.max(jnp.where(batch == b, xm, neg_inf), axis=0, keepdims=True)
    mx_rows.append(jnp.where(jnp.isfinite(col), col, 0.0))
  mx_ref[...] = jnp.concatenate(mx_rows, axis=0)


def _tc4(hout, g, keep, batch):
  return pl.pallas_call(
      _tc4_body,
      out_shape=[
          jax.ShapeDtypeStruct((NPAD, 128), jnp.float32),
          jax.ShapeDtypeStruct((B, 128), jnp.float32),
          jax.ShapeDtypeStruct((B, 128), jnp.float32),
      ],
  )(hout, g, keep, batch)


def _tc5_body(mean_ref, mx_ref, w1_ref, b1_ref, w2_ref, b2_ref, w3_ref,
              b3_ref, logits_ref, probs_ref):
  mean = mean_ref[0] + mean_ref[1] + mean_ref[2] + mean_ref[3]
  mx = mx_ref[0] + mx_ref[1] + mx_ref[2] + mx_ref[3]
  r = jnp.concatenate([mean, mx], axis=1)
  z = jnp.maximum(
      jnp.dot(r, w1_ref[...], preferred_element_type=jnp.float32)
      + b1_ref[...], 0.0)
  z = jnp.maximum(
      jnp.dot(z, w2_ref[...], preferred_element_type=jnp.float32)
      + b2_ref[...], 0.0)
  lg = jnp.dot(z, w3_ref[...], preferred_element_type=jnp.float32) \
      + b3_ref[...]
  logits_ref[...] = lg
  m = jnp.max(lg, axis=1, keepdims=True)
  e = jnp.exp(lg - m)
  probs_ref[...] = e / jnp.sum(e, axis=1, keepdims=True)


def _tc5(means, maxes, w1, b1, w2, b2, w3, b3):
  return pl.pallas_call(
      _tc5_body,
      out_shape=[
          jax.ShapeDtypeStruct((B, NCLS), jnp.float32),
          jax.ShapeDtypeStruct((B, NCLS), jnp.float32),
      ],
  )(means, maxes, w1, b1, w2, b2, w3, b3)


def kernel(x, edge_index, batch, params):
  P = params
  src = edge_index[0].astype(jnp.int32)
  dst = edge_index[1].astype(jnp.int32)

  xp = jnp.pad(x.astype(jnp.float32), ((0, NPAD - N), (0, 0)))
  pad_ar = jnp.arange(EPAD - E, dtype=jnp.int32)
  src_m = jnp.concatenate([src, N + (pad_ar % 128)])
  dst2d = jnp.concatenate([dst, pad_ar % NPAD]).reshape(ER, 128)
  batchp = jnp.pad(batch.astype(jnp.int32), (0, NPAD - N),
                   constant_values=B - 1)
  batch2 = batchp.reshape(NR, 128)
  batchc = batchp.reshape(NPAD, 1)
  keep = jnp.pad(jnp.ones((N,), jnp.float32), (0, NPAD - N))

  means = []
  maxes = []
  h = xp
  for j in (1, 2, 3, 4):
    src_m, degpart = _edge_update(src_m, dst2d, keep)
    nmaskc = keep.reshape(NPAD, 1)
    hs, dis = _tc1(h, P['W%d' % j], degpart.reshape(NSC, NPAD, 1), nmaskc)
    (mpart,) = _gather_agg(hs, src_m, dst2d)
    hout, t, u = _tc2(
        mpart, hs, dis, nmaskc,
        P['b%d' % j].reshape(1, 128),
        P['p%dWrel' % j].reshape(1, 128),
        P['p%dWroot' % j].reshape(1, 128))
    (spart,) = _edge_score(src_m, dst2d, t.reshape(NPAD))
    keep2, g2 = _tc3(
        spart.reshape(NSC, NR, 128),
        u.reshape(NR, 128),
        P['p%dbrel' % j].reshape(1, 1),
        batch2,
        keep.reshape(NR, 128))
    keep = keep2.reshape(NPAD)
    xg, mean_j, mx_j = _tc4(hout, g2.reshape(NPAD, 1), keep.reshape(NPAD, 1),
                            batchc)
    means.append(mean_j)
    maxes.append(mx_j)
    h = xg

  logits, probs = _tc5(
      jnp.stack(means), jnp.stack(maxes),
      P['L1W'], P['L1b'].reshape(1, HID),
      P['L2W'], P['L2b'].reshape(1, HID // 2),
      P['L3W'], P['L3b'].reshape(1, NCLS))
  return logits, probs
